# Initial kernel scaffold; baseline (speedup 1.0000x reference)
import jax
import jax.numpy as jnp
from jax.experimental import pallas as pl


def kernel(x, fc1_w, fc2_w, gc_w, gc_b, bn0_g, bn0_b, gcbn_g, gcbn_b, bn1_g, bn1_b):
    raise NotImplementedError("write your pallas kernel here")


# speedup vs baseline: 384.5955x; 384.5955x over previous
import functools

import jax
import jax.numpy as jnp
from jax import lax
from jax.experimental import pallas as pl
from jax.experimental.pallas import tpu as pltpu
from jax.experimental.pallas import tpu_sc as plsc

B, C, N, K = 2, 128, 4096, 16
BN = 256
NB = N // BN
EPS = 1e-5

NC_SC, NS_SC = 2, 16
NW = NC_SC * NS_SC
NODES_W = (B * N) // NW
CHUNK = 8
NCH = NODES_W // CHUNK

_HI = lax.Precision.HIGHEST


def _dot_nt(a, b, precision=_HI):
    return lax.dot_general(a, b, (((1,), (1,)), ((), ())),
                           preferred_element_type=jnp.float32,
                           precision=precision)


def _gelu(x):
    return x * 0.5 * (1.0 + lax.erf(x * 0.7071067811865476))


def _bn_rows(y, g, b):
    mean = jnp.mean(y, axis=0, keepdims=True)
    var = jnp.mean((y - mean) ** 2, axis=0, keepdims=True)
    return (y - mean) / jnp.sqrt(var + EPS) * g + b


def _prologue_body(x_ref, w_ref, g_ref, b_ref, h_ref):
    y = jnp.dot(x_ref[...], w_ref[...], preferred_element_type=jnp.float32,
                precision=lax.Precision.DEFAULT)
    h_ref[...] = _bn_rows(y, g_ref[...], b_ref[...])


def _knn_body(hb_ref, ha_ref, idx_ref):
    b = pl.program_id(0)
    hb = hb_ref[0]
    ha = ha_ref[0]
    sq = jnp.sum(ha * ha, axis=1, keepdims=True)
    sqb = jnp.sum(hb * hb, axis=1, keepdims=True)
    ones = jnp.ones((BN, 1), jnp.float32)
    inner = -2.0 * _dot_nt(hb, ha, precision=lax.Precision.DEFAULT)
    s = (sqb + inner) + _dot_nt(ones, sq)
    jcol = lax.broadcasted_iota(jnp.int32, (BN, N), 1)
    cols = []
    for _ in range(K):
        m = jnp.min(s, axis=1, keepdims=True)
        amin = jnp.min(jnp.where(s <= m, jcol, jnp.int32(N)),
                       axis=1, keepdims=True)
        cols.append(amin)
        s = jnp.where(jcol == amin, jnp.float32(jnp.inf), s)
    idx_ref[0] = jnp.concatenate(cols, axis=1) + b * N


def _epilogue_body(h_ref, mg_ref, x_ref, we_ref, wo_ref, gcb_ref,
                   gg_ref, gb_ref, b1g_ref, b1b_ref, w2_ref, out_ref):
    h = h_ref[...]
    xjm = mg_ref[...] - h
    g = (jnp.dot(h, we_ref[...], preferred_element_type=jnp.float32,
                 precision=lax.Precision.DEFAULT)
         + jnp.dot(xjm, wo_ref[...], preferred_element_type=jnp.float32,
                   precision=lax.Precision.DEFAULT)
         + gcb_ref[...])
    g = _bn_rows(g, gg_ref[...], gb_ref[...])
    g = _gelu(g)
    g = _gelu(g)
    g = _bn_rows(g, b1g_ref[...], b1b_ref[...])
    out_ref[...] = (jnp.dot(g, w2_ref[...], preferred_element_type=jnp.float32,
                            precision=lax.Precision.DEFAULT)
                    + x_ref[...])


@functools.cache
def _build_sc_gather_max():
    mesh = plsc.VectorSubcoreMesh(core_axis_name="c", subcore_axis_name="s",
                                  num_cores=NC_SC, num_subcores=NS_SC)

    @functools.partial(
        pl.kernel,
        out_type=jax.ShapeDtypeStruct((B * N, C), jnp.float32),
        mesh=mesh,
        scratch_types=[
            pltpu.VMEM((NCH, CHUNK * K), jnp.int32),
            pltpu.VMEM((CHUNK * K, C), jnp.float32),
            pltpu.VMEM((CHUNK, C), jnp.float32),
            pltpu.SemaphoreType.DMA,
        ],
    )
    def sc_gather_max(h_hbm, idx_hbm, out_hbm, idx_v, rows_v, out_v, sem):
        wid = lax.axis_index("s") * NC_SC + lax.axis_index("c")
        base = wid * NODES_W
        pltpu.sync_copy(idx_hbm.at[wid], idx_v)

        def chunk(ci, _):
            pltpu.async_copy(h_hbm.at[idx_v.at[ci]], rows_v, sem).wait()
            for n in range(CHUNK):
                for c8 in range(C // 16):
                    sl = pl.ds(c8 * 16, 16)
                    acc = rows_v[n * K, sl]
                    for k in range(1, K):
                        acc = jnp.maximum(acc, rows_v[n * K + k, sl])
                    out_v[n, sl] = acc
            pltpu.sync_copy(out_v, out_hbm.at[pl.ds(base + ci * CHUNK, CHUNK)])
            return 0

        lax.fori_loop(0, NCH, chunk, 0)

    return sc_gather_max


def _sc_gather_max(h, idx3):
    return _build_sc_gather_max()(h, idx3)


def _prologue(x_nc, fc1_wt, g, b):
    return pl.pallas_call(
        _prologue_body,
        out_shape=jax.ShapeDtypeStruct((B * N, C), jnp.float32),
    )(x_nc, fc1_wt, g, b)


def _knn(h3):
    return pl.pallas_call(
        _knn_body,
        grid=(B, NB),
        in_specs=[
            pl.BlockSpec((1, BN, C), lambda b, i: (b, i, 0)),
            pl.BlockSpec((1, N, C), lambda b, i: (b, 0, 0)),
        ],
        out_specs=pl.BlockSpec((1, BN, K), lambda b, i: (b, i, 0)),
        out_shape=jax.ShapeDtypeStruct((B, N, K), jnp.int32),
    )(h3, h3)


def _epilogue(h, mg, x_nc, we_t, wo_t, gcb, gg, gb, b1g, b1b, fc2_wt):
    return pl.pallas_call(
        _epilogue_body,
        out_shape=jax.ShapeDtypeStruct((B * N, C), jnp.float32),
    )(h, mg, x_nc, we_t, wo_t, gcb, gg, gb, b1g, b1b, fc2_wt)


def kernel(x, fc1_w, fc2_w, gc_w, gc_b, bn0_g, bn0_b, gcbn_g, gcbn_b,
           bn1_g, bn1_b):
    x_nc = x[:, :, :, 0].transpose(0, 2, 1).reshape(B * N, C)
    h = _prologue(x_nc, fc1_w.T, bn0_g.reshape(1, C), bn0_b.reshape(1, C))
    idx = _knn(h.reshape(B, N, C))
    mg = _sc_gather_max(h, idx.reshape(NW, NCH, CHUNK * K))
    out = _epilogue(h, mg, x_nc,
                    gc_w[:, 0::2].T, gc_w[:, 1::2].T, gc_b.reshape(1, C),
                    gcbn_g.reshape(1, C), gcbn_b.reshape(1, C),
                    bn1_g.reshape(1, C), bn1_b.reshape(1, C), fc2_w.T)
    return out.reshape(B, N, C).transpose(0, 2, 1)[:, :, :, None]
